# baseline (speedup 1.0000x reference)
import functools

import jax
import jax.numpy as jnp
from jax import lax
from jax.experimental import pallas as pl
from jax.experimental.pallas import tpu as pltpu
from jax.experimental.pallas import tpu_sc as plsc

B = 256
S = 197
D = 768
P = 8192
L = 8
K = 8

_EPS = 1e-12


_BB = 8


def _mean_norm_copy_body(x_ref, copy_ref, xn_ref):
    xb = x_ref[...]
    copy_ref[...] = xb
    mean = jnp.sum(xb, axis=1) * jnp.float32(1.0 / S)
    ss = jnp.sum(mean * mean, axis=1, keepdims=True)
    xn_ref[...] = mean * lax.rsqrt(jnp.maximum(ss, jnp.float32(_EPS)))


def _k_mean_norm_copy(x_embed):
    return pl.pallas_call(
        _mean_norm_copy_body,
        grid=(B // _BB,),
        in_specs=[pl.BlockSpec((_BB, S, D), lambda i: (i, 0, 0))],
        out_specs=[
            pl.BlockSpec((_BB, S, D), lambda i: (i, 0, 0)),
            pl.BlockSpec((_BB, D), lambda i: (i, 0)),
        ],
        out_shape=[
            jax.ShapeDtypeStruct((B, S, D), jnp.float32),
            jax.ShapeDtypeStruct((B, D), jnp.float32),
        ],
    )(x_embed)


_PB = 512


def _keynorm_body(k_ref, kn_ref):
    kb = k_ref[...]
    ss = jnp.sum(kb * kb, axis=1, keepdims=True)
    kn_ref[...] = kb * lax.rsqrt(jnp.maximum(ss, jnp.float32(_EPS)))


def _k_keynorm(prompt_key):
    return pl.pallas_call(
        _keynorm_body,
        grid=(P // _PB,),
        in_specs=[pl.BlockSpec((_PB, D), lambda i: (i, 0))],
        out_specs=pl.BlockSpec((_PB, D), lambda i: (i, 0)),
        out_shape=jax.ShapeDtypeStruct((P, D), jnp.float32),
    )(prompt_key)


_PC = 1024
_NPC = P // _PC


def _sim_topk_body(xn_ref, keys_ref, sim_ref, topk_ref, idx_ref, red_ref,
                   acc_ref):
    i = pl.program_id(0)
    ch = lax.dot_general(
        xn_ref[...], keys_ref[...],
        dimension_numbers=(((1,), (1,)), ((), ())),
        preferred_element_type=jnp.float32,
    )
    sim_ref[...] = ch
    acc_ref[:, pl.ds(i * _PC, _PC)] = ch

    @pl.when(i == _NPC - 1)
    def _():
        vals = []
        ids = []
        for _k in range(K):
            s = acc_ref[...]
            m = jnp.max(s, axis=1, keepdims=True)
            iota = lax.broadcasted_iota(jnp.int32, (B, P), 1)
            ik = jnp.min(jnp.where(s == m, iota, jnp.int32(P)),
                         axis=1, keepdims=True)
            vals.append(m)
            ids.append(ik)
            acc_ref[...] = jnp.where(iota == ik, jnp.float32(-jnp.inf), s)
        tv = jnp.concatenate(vals, axis=1)
        topk_ref[...] = tv
        idx_ref[...] = jnp.concatenate(ids, axis=1)
        red_ref[0, 0] = jnp.sum(tv) * jnp.float32(1.0 / B)


def _k_sim_topk(xn, pkn):
    return pl.pallas_call(
        _sim_topk_body,
        grid=(_NPC,),
        in_specs=[
            pl.BlockSpec((B, D), lambda i: (0, 0)),
            pl.BlockSpec((_PC, D), lambda i: (i, 0)),
        ],
        out_specs=[
            pl.BlockSpec((B, _PC), lambda i: (0, i)),
            pl.BlockSpec((B, K), lambda i: (0, 0)),
            pl.BlockSpec((B, K), lambda i: (0, 0)),
            pl.BlockSpec((1, 1), lambda i: (0, 0), memory_space=pltpu.SMEM),
        ],
        out_shape=[
            jax.ShapeDtypeStruct((B, P), jnp.float32),
            jax.ShapeDtypeStruct((B, K), jnp.float32),
            jax.ShapeDtypeStruct((B, K), jnp.int32),
            jax.ShapeDtypeStruct((1, 1), jnp.float32),
        ],
        scratch_shapes=[pltpu.VMEM((B, P), jnp.float32)],
    )(xn, pkn)


_NW = 32
_RW = (B * K) // _NW
_CH = 4
_NCH = _RW // _CH
_PD = L * D


def _sc_gather_body(prompt_hbm, pkn_hbm, idx1_hbm, idx4_hbm,
                    out1, out2, outk,
                    idx_v, idx4_v, bufs, keys_v, sem0, sem1, semk):
    c = lax.axis_index("c")
    s = lax.axis_index("s")
    wid = s * 2 + c
    base = wid * _RW
    pltpu.sync_copy(idx1_hbm.at[pl.ds(base, _RW)], idx_v)
    pltpu.sync_copy(idx4_hbm.at[pl.ds(wid * _NCH, _NCH)], idx4_v)

    kd = pltpu.make_async_copy(pkn_hbm.at[idx_v], keys_v, semk)
    kd.start()

    sems = [sem0, sem1]

    def _fire(ci):
        d = pltpu.make_async_copy(
            prompt_hbm.at[idx4_v.at[ci]], bufs.at[ci % 2], sems[ci % 2])
        d.start()
        return d

    descs = [_fire(0)]
    for ci in range(_NCH):
        if ci + 1 < _NCH:
            descs.append(_fire(ci + 1))
        descs[ci].wait()
        r0 = base + ci * _CH
        pltpu.sync_copy(bufs.at[ci % 2], out1.at[pl.ds(r0, _CH)])
        pltpu.sync_copy(bufs.at[ci % 2], out2.at[pl.ds(r0, _CH)])

    kd.wait()
    pltpu.sync_copy(keys_v, outk.at[pl.ds(base, _RW)])


@functools.cache
def _sc_gather():
    return pl.kernel(
        _sc_gather_body,
        mesh=plsc.VectorSubcoreMesh(core_axis_name="c", subcore_axis_name="s"),
        out_type=(
            jax.ShapeDtypeStruct((B * K, _PD), jnp.float32),
            jax.ShapeDtypeStruct((B * K, _PD), jnp.float32),
            jax.ShapeDtypeStruct((B * K, D), jnp.float32),
        ),
        scratch_types=[
            pltpu.VMEM((_RW,), jnp.int32),
            pltpu.VMEM((_NCH, _CH), jnp.int32),
            pltpu.VMEM((2, _CH, _PD), jnp.float32),
            pltpu.VMEM((_RW, D), jnp.float32),
            pltpu.SemaphoreType.DMA,
            pltpu.SemaphoreType.DMA,
            pltpu.SemaphoreType.DMA,
        ],
    )


def kernel(x_embed, prompt, prompt_key, is_training):
    pe, xn = _k_mean_norm_copy(x_embed)
    pkn = _k_keynorm(prompt_key)
    sim, topk, idx, red = _k_sim_topk(xn, pkn)
    idx_flat = idx.reshape(B * K)
    g1, g2, gk = _sc_gather()(
        prompt.reshape(P, _PD), pkn, idx_flat, idx_flat.reshape(-1, _CH))
    batched_prompt = g1.reshape(B, K * L, D)
    selected_prompt = g2.reshape(B, K * L * D)
    selected_key = gk.reshape(B, K * D)
    return (batched_prompt, red[0, 0], sim, topk, idx, pkn, xn,
            selected_key, selected_prompt, pe)
